# baseline (speedup 1.0000x reference)
import functools

import jax
import jax.numpy as jnp
from jax import lax
from jax.experimental import pallas as pl
from jax.experimental.pallas import tpu as pltpu
from jax.experimental.pallas import tpu_sc as plsc

EMB = 4096
NE = 64
K = 8
NT = 8192
M_BLK = 1024

NCHUNK = 4
CT = NT // NCHUNK
NW = 32
TPW = CT // NW
GRP = TPW // 16
GI = 4


def _mm_block(x_ref, w_ref, scores_ref):
    scores_ref[...] = jax.lax.dot_general(
        x_ref[...], w_ref[...], (((1,), (1,)), ((), ())),
        preferred_element_type=jnp.float32,
    )


def _gate_scores(x, w):
    grid = (CT // M_BLK,)
    return pl.pallas_call(
        _mm_block,
        grid=grid,
        in_specs=[
            pl.BlockSpec((M_BLK, EMB), lambda i: (i, 0)),
            pl.BlockSpec((NE, EMB), lambda i: (0, 0)),
        ],
        out_specs=pl.BlockSpec((M_BLK, NE), lambda i: (i, 0)),
        out_shape=jax.ShapeDtypeStruct((CT, NE), jnp.float32),
    )(x, w)


def _topk_body(scores_hbm, probs_hbm, idx_hbm, svmem, pvmem, ivmem):
    wid = lax.axis_index("s") * 2 + lax.axis_index("c")
    pltpu.sync_copy(scores_hbm.at[pl.ds(wid * TPW * NE, TPW * NE)], svmem)

    lane = lax.iota(jnp.int32, 16)
    lane64 = lane * NE
    lane8 = lane * K
    m6 = jnp.int32(NE - 1)
    neg_inf = jnp.full((16,), -jnp.inf, jnp.float32)

    def set_body(si, _):
        rb64 = [(si * GI + k) * 16 * NE + lane64 for k in range(GI)]
        rb8 = [(si * GI + k) * 16 * K + lane8 for k in range(GI)]

        def exp_body(e, ts):
            ts = list(ts)
            tie_base = jnp.int32(NE - 1) - e
            for k in range(GI):
                v = plsc.load_gather(svmem, [rb64[k] + e])
                i = plsc.bitcast(v, jnp.int32)
                sgn = lax.shift_right_arithmetic(i, 31)
                key = plsc.bitcast((i & ~m6) | (tie_base ^ (sgn & m6)), jnp.float32)
                for j in range(K):
                    t = ts[k * K + j]
                    ts[k * K + j] = jnp.maximum(t, key)
                    key = jnp.minimum(t, key)
            return tuple(ts)

        ts = lax.fori_loop(0, NE, exp_body, (neg_inf,) * (GI * K), unroll=4)

        for k in range(GI):
            tb = [plsc.bitcast(ts[k * K + j], jnp.int32) for j in range(K)]
            vals = [plsc.bitcast(b & ~m6, jnp.float32) for b in tb]
            es = [jnp.exp(v - vals[0]) for v in vals]
            tot = es[0]
            for j in range(1, K):
                tot = tot + es[j]
            for j in range(K):
                idx_j = (tb[j] & m6) ^ (~lax.shift_right_arithmetic(tb[j], 31) & m6)
                plsc.store_scatter(pvmem, [rb8[k] + j], es[j] / tot)
                plsc.store_scatter(ivmem, [rb8[k] + j], idx_j)
        return 0

    lax.fori_loop(0, GRP // GI, set_body, 0)
    pltpu.sync_copy(pvmem, probs_hbm.at[pl.ds(wid * TPW * K, TPW * K)])
    pltpu.sync_copy(ivmem, idx_hbm.at[pl.ds(wid * TPW * K, TPW * K)])


_topk_sc = functools.partial(
    pl.kernel,
    out_type=[
        jax.ShapeDtypeStruct((CT * K,), jnp.float32),
        jax.ShapeDtypeStruct((CT * K,), jnp.int32),
    ],
    mesh=plsc.VectorSubcoreMesh(core_axis_name="c", subcore_axis_name="s"),
    compiler_params=pltpu.CompilerParams(needs_layout_passes=False),
    scratch_types=[
        pltpu.VMEM((TPW * NE,), jnp.float32),
        pltpu.VMEM((TPW * K,), jnp.float32),
        pltpu.VMEM((TPW * K,), jnp.int32),
    ],
)(_topk_body)


@jax.jit
def kernel(x, W_gate):
    ps, isx, ss = [], [], []
    for q in range(NCHUNK):
        scores_q = _gate_scores(jax.lax.slice(x, (q * CT, 0), ((q + 1) * CT, EMB)), W_gate)
        pq, iq = _topk_sc(scores_q.reshape(CT * NE))
        ps.append(pq.reshape(CT, K))
        isx.append(iq.reshape(CT, K))
        ss.append(scores_q)
    return (jnp.concatenate(ps), jnp.concatenate(isx), jnp.concatenate(ss))


# speedup vs baseline: 1.9984x; 1.9984x over previous
import jax
import jax.numpy as jnp
from jax.experimental import pallas as pl

EMB = 4096
NE = 64
K = 8
NT = 8192
M_BLK = 1024
KS = 4
KB = EMB // KS
SUB = 128


def _router_block(x_ref, w_ref, probs_ref, idx_ref, scores_ref):
    kstep = pl.program_id(1)
    w = w_ref[...]
    part = jax.lax.dot_general(
        x_ref[...], w, (((1,), (1,)), ((), ())), preferred_element_type=jnp.float32
    )

    @pl.when(kstep == 0)
    def _():
        scores_ref[...] = part

    @pl.when(kstep != 0)
    def _():
        scores_ref[...] = scores_ref[...] + part

    @pl.when(kstep == KS - 1)
    def _():
        cols63 = jnp.int32(NE - 1) - jax.lax.broadcasted_iota(jnp.int32, (SUB, NE), 1)
        m6 = jnp.int32(NE - 1)
        neg_inf = jnp.float32(-jnp.inf)
        for c in range(M_BLK // SUB):
            s = scores_ref[c * SUB:(c + 1) * SUB, :]
            i = jax.lax.bitcast_convert_type(s, jnp.int32)
            sgn = jax.lax.shift_right_arithmetic(i, 31)
            tie = cols63 ^ (sgn & m6)
            key = jax.lax.bitcast_convert_type((i & ~m6) | tie, jnp.float32)
            tops = []
            for _ in range(K):
                m = jnp.max(key, axis=1, keepdims=True)
                tops.append(m)
                key = jnp.where(key == m, neg_inf, key)
            tk = jnp.concatenate(tops, axis=1)
            tb = jax.lax.bitcast_convert_type(tk, jnp.int32)
            tsgn = jax.lax.shift_right_arithmetic(tb, 31)
            top_idx = (tb & m6) ^ (~tsgn & m6)
            vals = jax.lax.bitcast_convert_type(tb & ~m6, jnp.float32)
            e = jnp.exp(vals - vals[:, 0:1])
            probs = e / jnp.sum(e, axis=1, keepdims=True)
            probs_ref[c * SUB:(c + 1) * SUB, :] = probs
            idx_ref[c * SUB:(c + 1) * SUB, :] = top_idx


@jax.jit
def kernel(x, W_gate):
    grid = (NT // M_BLK, KS)
    probs, idx, scores = pl.pallas_call(
        _router_block,
        grid=grid,
        in_specs=[
            pl.BlockSpec((M_BLK, KB), lambda i, k: (i, k)),
            pl.BlockSpec((NE, KB), lambda i, k: (0, k)),
        ],
        out_specs=[
            pl.BlockSpec((M_BLK, K), lambda i, k: (i, 0)),
            pl.BlockSpec((M_BLK, K), lambda i, k: (i, 0)),
            pl.BlockSpec((M_BLK, NE), lambda i, k: (i, 0)),
        ],
        out_shape=[
            jax.ShapeDtypeStruct((NT, K), jnp.float32),
            jax.ShapeDtypeStruct((NT, K), jnp.int32),
            jax.ShapeDtypeStruct((NT, NE), jnp.float32),
        ],
    )(x, W_gate)
    return (probs, idx, scores)


# speedup vs baseline: 3.0847x; 1.5436x over previous
import jax
import jax.numpy as jnp
from jax import lax
from jax.experimental import pallas as pl
from jax.experimental.pallas import tpu as pltpu

EMB = 4096
NE = 64
K = 8
NT = 8192
CR = 512
NBUF = 4
NCH = NT // CR
SUB = 128


def _router(x_hbm, w_ref, probs_ref, idx_ref, scores_ref, xbuf, sem):
    w = w_ref[...]
    cols63 = jnp.int32(NE - 1) - jax.lax.broadcasted_iota(jnp.int32, (SUB, NE), 1)
    m6 = jnp.int32(NE - 1)
    neg_inf = jnp.float32(-jnp.inf)

    def copy(c):
        b = lax.rem(c, NBUF)
        return pltpu.make_async_copy(
            x_hbm.at[pl.ds(c * CR, CR), :],
            xbuf.at[pl.ds(b * CR, CR), :],
            sem.at[b],
        )

    for c in range(NBUF - 1):
        copy(c).start()

    def chunk_body(c, _):
        copy(c).wait()
        boff = lax.rem(c, NBUF) * CR
        base = c * CR
        for s in range(CR // SUB):
            xs = xbuf[pl.ds(boff + s * SUB, SUB), :]
            sc = jax.lax.dot_general(
                xs, w, (((1,), (1,)), ((), ())), preferred_element_type=jnp.float32
            )
            scores_ref[pl.ds(base + s * SUB, SUB), :] = sc
            i = jax.lax.bitcast_convert_type(sc, jnp.int32)
            sgn = jax.lax.shift_right_arithmetic(i, 31)
            tie = cols63 ^ (sgn & m6)
            key = jax.lax.bitcast_convert_type((i & ~m6) | tie, jnp.float32)
            tops = []
            for _ in range(K):
                m = jnp.max(key, axis=1, keepdims=True)
                tops.append(m)
                key = jnp.where(key == m, neg_inf, key)
            tk = jnp.concatenate(tops, axis=1)
            tb = jax.lax.bitcast_convert_type(tk, jnp.int32)
            tsgn = jax.lax.shift_right_arithmetic(tb, 31)
            top_idx = (tb & m6) ^ (~tsgn & m6)
            vals = jax.lax.bitcast_convert_type(tb & ~m6, jnp.float32)
            e = jnp.exp(vals - vals[:, 0:1])
            probs = e / jnp.sum(e, axis=1, keepdims=True)
            probs_ref[pl.ds(base + s * SUB, SUB), :] = probs
            idx_ref[pl.ds(base + s * SUB, SUB), :] = top_idx

        @pl.when(c + NBUF - 1 < NCH)
        def _():
            copy(c + NBUF - 1).start()

        return 0

    lax.fori_loop(0, NCH, chunk_body, 0)


@jax.jit
def kernel(x, W_gate):
    probs, idx, scores = pl.pallas_call(
        _router,
        in_specs=[
            pl.BlockSpec(memory_space=pl.ANY),
            pl.BlockSpec(memory_space=pltpu.VMEM),
        ],
        out_specs=[
            pl.BlockSpec(memory_space=pltpu.VMEM),
            pl.BlockSpec(memory_space=pltpu.VMEM),
            pl.BlockSpec(memory_space=pltpu.VMEM),
        ],
        out_shape=[
            jax.ShapeDtypeStruct((NT, K), jnp.float32),
            jax.ShapeDtypeStruct((NT, K), jnp.int32),
            jax.ShapeDtypeStruct((NT, NE), jnp.float32),
        ],
        scratch_shapes=[
            pltpu.VMEM((NBUF * CR, EMB), jnp.float32),
            pltpu.SemaphoreType.DMA((NBUF,)),
        ],
    )(x, W_gate)
    return (probs, idx, scores)


# speedup vs baseline: 3.2117x; 1.0412x over previous
import jax
import jax.numpy as jnp
from jax.experimental import pallas as pl

EMB = 4096
NE = 64
K = 8
NT = 8192
M_BLK = 1024
SUB = 256


def _router_block(x_ref, w_ref, probs_ref, idx_ref, scores_ref):
    w = w_ref[...]
    cols63 = jnp.int32(NE - 1) - jax.lax.broadcasted_iota(jnp.int32, (SUB, NE), 1)
    m6 = jnp.int32(NE - 1)
    neg_inf = jnp.float32(-jnp.inf)
    for c in range(M_BLK // SUB):
        x = x_ref[c * SUB:(c + 1) * SUB, :]
        s = jax.lax.dot_general(
            x, w, (((1,), (1,)), ((), ())), preferred_element_type=jnp.float32
        )
        scores_ref[c * SUB:(c + 1) * SUB, :] = s
        i = jax.lax.bitcast_convert_type(s, jnp.int32)
        sgn = jax.lax.shift_right_arithmetic(i, 31)
        tie = cols63 ^ (sgn & m6)
        key = jax.lax.bitcast_convert_type((i & ~m6) | tie, jnp.float32)
        tops = []
        for _ in range(K):
            m = jnp.max(key, axis=1, keepdims=True)
            tops.append(m)
            key = jnp.where(key == m, neg_inf, key)
        tk = jnp.concatenate(tops, axis=1)
        tb = jax.lax.bitcast_convert_type(tk, jnp.int32)
        tsgn = jax.lax.shift_right_arithmetic(tb, 31)
        top_idx = (tb & m6) ^ (~tsgn & m6)
        vals = jax.lax.bitcast_convert_type(tb & ~m6, jnp.float32)
        e = jnp.exp(vals - vals[:, 0:1])
        probs = e / jnp.sum(e, axis=1, keepdims=True)
        probs_ref[c * SUB:(c + 1) * SUB, :] = probs
        idx_ref[c * SUB:(c + 1) * SUB, :] = top_idx


@jax.jit
def kernel(x, W_gate):
    grid = (NT // M_BLK,)
    probs, idx, scores = pl.pallas_call(
        _router_block,
        grid=grid,
        in_specs=[
            pl.BlockSpec((M_BLK, EMB), lambda i: (i, 0)),
            pl.BlockSpec((NE, EMB), lambda i: (0, 0)),
        ],
        out_specs=[
            pl.BlockSpec((M_BLK, K), lambda i: (i, 0)),
            pl.BlockSpec((M_BLK, K), lambda i: (i, 0)),
            pl.BlockSpec((M_BLK, NE), lambda i: (i, 0)),
        ],
        out_shape=[
            jax.ShapeDtypeStruct((NT, K), jnp.float32),
            jax.ShapeDtypeStruct((NT, K), jnp.int32),
            jax.ShapeDtypeStruct((NT, NE), jnp.float32),
        ],
    )(x, W_gate)
    return (probs, idx, scores)
